# baseline (speedup 1.0000x reference)
import jax
import jax.numpy as jnp
from jax.experimental import pallas as pl
from jax.experimental.pallas import tpu as pltpu

NUM_EMBEDDINGS = 2048
EMBEDDING_DIM = 256
COMMITMENT_COST = 0.25

_ROWS = 16 * 2048
_BLK = 512
_NBLK = _ROWS // _BLK


def _argmin_kernel(x_ref, w_ref, idx_ref):
    x = x_ref[...]
    w = w_ref[...]
    mm = jax.lax.dot_general(
        x, w, (((1,), (1,)), ((), ())), preferred_element_type=jnp.float32)
    xsq = jnp.sum(x * x, axis=1, keepdims=True)
    wsq = jnp.sum(w * w, axis=1)
    d = (xsq + wsq[None, :]) - 2.0 * mm
    minval = jnp.min(d, axis=1, keepdims=True)
    ids = jax.lax.broadcasted_iota(jnp.int32, d.shape, 1)
    idx = jnp.min(jnp.where(d == minval, ids, NUM_EMBEDDINGS), axis=1)
    idx_ref[0, 0, :] = idx


def _compute_indices(flat_input, W):
    idx3 = pl.pallas_call(
        _argmin_kernel,
        grid=(_NBLK,),
        in_specs=[
            pl.BlockSpec((_BLK, EMBEDDING_DIM), lambda i: (i, 0)),
            pl.BlockSpec((NUM_EMBEDDINGS, EMBEDDING_DIM), lambda i: (0, 0)),
        ],
        out_specs=pl.BlockSpec((1, 1, _BLK), lambda i: (i, 0, 0)),
        out_shape=jax.ShapeDtypeStruct((_NBLK, 1, _BLK), jnp.int32),
    )(flat_input, W)
    return idx3.reshape(_ROWS)


def kernel(inputs, W):
    N, width, T = inputs.shape
    inputs_p = jnp.transpose(inputs, (0, 2, 1))
    flat_input = inputs_p.reshape(-1, width)

    encoding_indices = _compute_indices(flat_input, W)

    encodings = jax.nn.one_hot(encoding_indices, NUM_EMBEDDINGS,
                               dtype=jnp.float32)
    quantized = jnp.matmul(encodings, W).reshape(N, T, width)

    reset_ratio = jnp.zeros((1,), dtype=jnp.float32)

    e_latent_loss = jnp.mean((jax.lax.stop_gradient(quantized) - inputs_p) ** 2)
    q_latent_loss = jnp.mean((quantized - jax.lax.stop_gradient(inputs_p)) ** 2)
    loss = q_latent_loss + COMMITMENT_COST * e_latent_loss

    quantized_st = inputs_p + jax.lax.stop_gradient(quantized - inputs_p)
    quantized_out = jnp.transpose(quantized_st, (0, 2, 1))

    avg_probs = jnp.mean(encodings, axis=0)
    perplexity = jnp.exp(-jnp.sum(avg_probs * jnp.log(avg_probs + 1e-10)))

    return (loss, quantized_out, perplexity, reset_ratio, encodings)


# speedup vs baseline: 1.0449x; 1.0449x over previous
import functools

import jax
import jax.numpy as jnp
from jax import lax
from jax.experimental import pallas as pl
from jax.experimental.pallas import tpu as pltpu
from jax.experimental.pallas import tpu_sc as plsc

NUM_EMBEDDINGS = 2048
EMBEDDING_DIM = 256
COMMITMENT_COST = 0.25

_N = 16
_T = 2048
_ROWS = _N * _T
_BLK = 512
_NBLK = _ROWS // _BLK

_NC = 2
_NS = 16
_NW = _NC * _NS
_RPW = _ROWS // _NW
_SUB = 128
_NCHUNK = _RPW // _SUB


def _dist_argmin_kernel(x_ref, w_ref, idx_ref, oh_ref, loss_ref, perp_ref,
                        wsq_ref, cnt_ref, acc_ref):
    i = pl.program_id(0)
    x = x_ref[...]
    w = w_ref[...]

    @pl.when(i == 0)
    def _():
        wsq_ref[...] = jnp.sum(w * w, axis=1)
        cnt_ref[...] = jnp.zeros((8, NUM_EMBEDDINGS), jnp.float32)
        acc_ref[0] = 0.0

    mm = jax.lax.dot_general(
        x, w, (((1,), (1,)), ((), ())), preferred_element_type=jnp.float32)
    xsq = jnp.sum(x * x, axis=1, keepdims=True)
    d = (xsq + wsq_ref[...][None, :]) - 2.0 * mm
    minval = jnp.min(d, axis=1, keepdims=True)
    ids = jax.lax.broadcasted_iota(jnp.int32, d.shape, 1)
    idx = jnp.min(jnp.where(d == minval, ids, NUM_EMBEDDINGS), axis=1)
    idx_ref[0, 0, :] = idx
    oh = jnp.where(ids == idx[:, None], 1.0, 0.0)
    oh_ref[...] = oh

    cnt_ref[...] += jax.lax.dot_general(
        jnp.ones((8, _BLK), jnp.float32), oh, (((1,), (0,)), ((), ())),
        preferred_element_type=jnp.float32)
    acc_ref[0] += jnp.sum(minval)

    @pl.when(i == _NBLK - 1)
    def _():
        loss_ref[...] = jnp.full(
            (1, 1), (1.0 + COMMITMENT_COST) / (_ROWS * EMBEDDING_DIM)) \
            * acc_ref[0]
        p = cnt_ref[0:1, :] * (1.0 / _ROWS)
        perp_ref[...] = jnp.exp(
            -jnp.sum(p * jnp.log(p + 1e-10), keepdims=True))


def _dist_argmin(flat_input, W):
    return pl.pallas_call(
        _dist_argmin_kernel,
        grid=(_NBLK,),
        in_specs=[
            pl.BlockSpec((_BLK, EMBEDDING_DIM), lambda i: (i, 0)),
            pl.BlockSpec((NUM_EMBEDDINGS, EMBEDDING_DIM), lambda i: (0, 0)),
        ],
        out_specs=[
            pl.BlockSpec((1, 1, _BLK), lambda i: (i, 0, 0)),
            pl.BlockSpec((_BLK, NUM_EMBEDDINGS), lambda i: (i, 0)),
            pl.BlockSpec((1, 1), lambda i: (0, 0)),
            pl.BlockSpec((1, 1), lambda i: (0, 0)),
        ],
        out_shape=[
            jax.ShapeDtypeStruct((_NBLK, 1, _BLK), jnp.int32),
            jax.ShapeDtypeStruct((_ROWS, NUM_EMBEDDINGS), jnp.float32),
            jax.ShapeDtypeStruct((1, 1), jnp.float32),
            jax.ShapeDtypeStruct((1, 1), jnp.float32),
        ],
        scratch_shapes=[
            pltpu.VMEM((NUM_EMBEDDINGS,), jnp.float32),
            pltpu.VMEM((8, NUM_EMBEDDINGS), jnp.float32),
            pltpu.SMEM((1,), jnp.float32),
        ],
    )(flat_input, W)


def _sc_gather_body(idx_hbm, w_hbm, quant_hbm, idx_c, rows_v, sem):
    wid = lax.axis_index("s") * _NC + lax.axis_index("c")
    base = wid * _RPW
    for c in range(_NCHUNK):
        pltpu.sync_copy(idx_hbm.at[pl.ds(base + c * _SUB, _SUB)], idx_c)
        pltpu.async_copy(w_hbm.at[idx_c], rows_v, sem).wait()
        pltpu.sync_copy(rows_v, quant_hbm.at[pl.ds(base + c * _SUB, _SUB)])


def _sc_gather(encoding_indices, W):
    mesh = plsc.VectorSubcoreMesh(core_axis_name="c", subcore_axis_name="s")
    fn = functools.partial(
        pl.kernel,
        mesh=mesh,
        out_type=jax.ShapeDtypeStruct((_ROWS, EMBEDDING_DIM), jnp.float32),
        scratch_types=[
            pltpu.VMEM((_SUB,), jnp.int32),
            pltpu.VMEM((_SUB, EMBEDDING_DIM), jnp.float32),
            pltpu.SemaphoreType.DMA,
        ],
    )(_sc_gather_body)
    return fn(encoding_indices, W)


def _transpose_kernel(q_ref, out_ref):
    out_ref[0] = q_ref[...].T


def _transpose(quantized):
    return pl.pallas_call(
        _transpose_kernel,
        grid=(_N, _T // _BLK),
        in_specs=[
            pl.BlockSpec((_BLK, EMBEDDING_DIM),
                         lambda n, tb: (n * (_T // _BLK) + tb, 0)),
        ],
        out_specs=pl.BlockSpec((1, EMBEDDING_DIM, _BLK),
                               lambda n, tb: (n, 0, tb)),
        out_shape=jax.ShapeDtypeStruct((_N, EMBEDDING_DIM, _T), jnp.float32),
    )(quantized)


def kernel(inputs, W):
    N, width, T = inputs.shape
    inputs_p = jnp.transpose(inputs, (0, 2, 1))
    flat_input = inputs_p.reshape(-1, width)

    idx3, encodings, loss, perp = _dist_argmin(flat_input, W)
    encoding_indices = idx3.reshape(_ROWS)

    quantized = _sc_gather(encoding_indices, W)
    quantized_out = _transpose(quantized)

    loss = loss.reshape(())
    perplexity = perp.reshape(())
    reset_ratio = jnp.zeros((1,), dtype=jnp.float32)

    return (loss, quantized_out, perplexity, reset_ratio, encodings)


# speedup vs baseline: 1.0486x; 1.0035x over previous
import functools

import jax
import jax.numpy as jnp
from jax import lax
from jax.experimental import pallas as pl
from jax.experimental.pallas import tpu as pltpu
from jax.experimental.pallas import tpu_sc as plsc

NUM_EMBEDDINGS = 2048
EMBEDDING_DIM = 256
COMMITMENT_COST = 0.25

_N = 16
_T = 2048
_ROWS = _N * _T
_BLK = 512
_NBLK = _ROWS // _BLK

_NC = 2
_NS = 16
_NW = _NC * _NS
_RPW = _ROWS // _NW
_SUB = 128
_NCHUNK = _RPW // _SUB


def _dist_argmin_kernel(x_ref, w_ref, idx_ref, oh_ref, loss_ref, perp_ref,
                        wsq_ref, cnt_ref, acc_ref):
    i = pl.program_id(0)
    x = x_ref[...]
    w = w_ref[...]

    @pl.when(i == 0)
    def _():
        wsq_ref[...] = jnp.sum(w * w, axis=1)
        cnt_ref[...] = jnp.zeros((8, NUM_EMBEDDINGS), jnp.float32)
        acc_ref[0] = 0.0

    mm = jax.lax.dot_general(
        x, w, (((1,), (1,)), ((), ())), preferred_element_type=jnp.float32)
    xsq = jnp.sum(x * x, axis=1, keepdims=True)
    d = (xsq + wsq_ref[...][None, :]) - 2.0 * mm
    minval = jnp.min(d, axis=1, keepdims=True)
    ids = jax.lax.broadcasted_iota(jnp.int32, d.shape, 1)
    idx = jnp.min(jnp.where(d == minval, ids, NUM_EMBEDDINGS), axis=1)
    idx_ref[0, 0, :] = idx
    oh = jnp.where(ids == idx[:, None], 1.0, 0.0)
    oh_ref[...] = oh

    cnt_ref[...] += jax.lax.dot_general(
        jnp.ones((8, _BLK), jnp.float32), oh, (((1,), (0,)), ((), ())),
        preferred_element_type=jnp.float32)
    acc_ref[0] += jnp.sum(minval)

    @pl.when(i == _NBLK - 1)
    def _():
        loss_ref[...] = jnp.full(
            (1, 1), (1.0 + COMMITMENT_COST) / (_ROWS * EMBEDDING_DIM)) \
            * acc_ref[0]
        p = cnt_ref[0:1, :] * (1.0 / _ROWS)
        perp_ref[...] = jnp.exp(
            -jnp.sum(p * jnp.log(p + 1e-10), keepdims=True))


def _dist_argmin(flat_input, W):
    return pl.pallas_call(
        _dist_argmin_kernel,
        grid=(_NBLK,),
        in_specs=[
            pl.BlockSpec((_BLK, EMBEDDING_DIM), lambda i: (i, 0)),
            pl.BlockSpec((NUM_EMBEDDINGS, EMBEDDING_DIM), lambda i: (0, 0)),
        ],
        out_specs=[
            pl.BlockSpec((1, 1, _BLK), lambda i: (i, 0, 0)),
            pl.BlockSpec((_BLK, NUM_EMBEDDINGS), lambda i: (i, 0)),
            pl.BlockSpec((1, 1), lambda i: (0, 0)),
            pl.BlockSpec((1, 1), lambda i: (0, 0)),
        ],
        out_shape=[
            jax.ShapeDtypeStruct((_NBLK, 1, _BLK), jnp.int32),
            jax.ShapeDtypeStruct((_ROWS, NUM_EMBEDDINGS), jnp.float32),
            jax.ShapeDtypeStruct((1, 1), jnp.float32),
            jax.ShapeDtypeStruct((1, 1), jnp.float32),
        ],
        scratch_shapes=[
            pltpu.VMEM((NUM_EMBEDDINGS,), jnp.float32),
            pltpu.VMEM((8, NUM_EMBEDDINGS), jnp.float32),
            pltpu.SMEM((1,), jnp.float32),
        ],
    )(flat_input, W)


def _sc_gather_body(idx_hbm, w_hbm, quant_hbm, idx_v, rows0, rows1, s0, s1):
    wid = lax.axis_index("s") * _NC + lax.axis_index("c")
    base = wid * _RPW
    pltpu.sync_copy(idx_hbm.at[wid], idx_v)
    rows = (rows0, rows1)
    sems = (s0, s1)
    copies = [None, None]
    copies[0] = pltpu.async_copy(w_hbm.at[idx_v.at[0]], rows0, s0)
    for c in range(_NCHUNK):
        if c + 1 < _NCHUNK:
            copies[(c + 1) % 2] = pltpu.async_copy(
                w_hbm.at[idx_v.at[c + 1]], rows[(c + 1) % 2],
                sems[(c + 1) % 2])
        copies[c % 2].wait()
        pltpu.sync_copy(rows[c % 2],
                        quant_hbm.at[pl.ds(base + c * _SUB, _SUB)])


def _sc_gather(encoding_indices, W):
    mesh = plsc.VectorSubcoreMesh(core_axis_name="c", subcore_axis_name="s")
    fn = functools.partial(
        pl.kernel,
        mesh=mesh,
        out_type=jax.ShapeDtypeStruct((_ROWS, EMBEDDING_DIM), jnp.float32),
        scratch_types=[
            pltpu.VMEM((_NCHUNK, _SUB), jnp.int32),
            pltpu.VMEM((_SUB, EMBEDDING_DIM), jnp.float32),
            pltpu.VMEM((_SUB, EMBEDDING_DIM), jnp.float32),
            pltpu.SemaphoreType.DMA,
            pltpu.SemaphoreType.DMA,
        ],
    )(_sc_gather_body)
    return fn(encoding_indices.reshape(_NW, _NCHUNK, _SUB), W)


def _transpose_kernel(q_ref, out_ref):
    out_ref[0] = q_ref[...].T


def _transpose(quantized):
    return pl.pallas_call(
        _transpose_kernel,
        grid=(_N, _T // _BLK),
        in_specs=[
            pl.BlockSpec((_BLK, EMBEDDING_DIM),
                         lambda n, tb: (n * (_T // _BLK) + tb, 0)),
        ],
        out_specs=pl.BlockSpec((1, EMBEDDING_DIM, _BLK),
                               lambda n, tb: (n, 0, tb)),
        out_shape=jax.ShapeDtypeStruct((_N, EMBEDDING_DIM, _T), jnp.float32),
    )(quantized)


def kernel(inputs, W):
    N, width, T = inputs.shape
    inputs_p = jnp.transpose(inputs, (0, 2, 1))
    flat_input = inputs_p.reshape(-1, width)

    idx3, encodings, loss, perp = _dist_argmin(flat_input, W)
    encoding_indices = idx3.reshape(_ROWS)

    quantized = _sc_gather(encoding_indices, W)
    quantized_out = _transpose(quantized)

    loss = loss.reshape(())
    perplexity = perp.reshape(())
    reset_ratio = jnp.zeros((1,), dtype=jnp.float32)

    return (loss, quantized_out, perplexity, reset_ratio, encodings)


# speedup vs baseline: 1.1088x; 1.0574x over previous
import functools

import jax
import jax.numpy as jnp
from jax import lax
from jax.experimental import pallas as pl
from jax.experimental.pallas import tpu as pltpu
from jax.experimental.pallas import tpu_sc as plsc

NUM_EMBEDDINGS = 2048
EMBEDDING_DIM = 256
COMMITMENT_COST = 0.25

_N = 16
_T = 2048
_ROWS = _N * _T
_BLK = 512
_TB = _T // _BLK
_NBLK = _ROWS // _BLK

_NC = 2
_NS = 16
_NW = _NC * _NS
_RPW = _ROWS // _NW
_SUB = 128
_NCHUNK = _RPW // _SUB


def _dist_argmin_kernel(x_ref, w2_ref, idx_ref, oh_ref, loss_ref, perp_ref,
                        wsq_ref, cnt_ref, acc_ref):
    i = pl.program_id(0)
    xb = x_ref[0]
    w2 = w2_ref[...]

    @pl.when(i == 0)
    def _():
        wsq_ref[...] = (jnp.sum(w2 * w2, axis=1) * 0.25)[None, :]
        cnt_ref[...] = jnp.zeros((8, NUM_EMBEDDINGS), jnp.float32)
        acc_ref[0] = 0.0

    mm2 = jax.lax.dot_general(
        xb, w2, (((0,), (1,)), ((), ())),
        preferred_element_type=jnp.float32)
    xsq = jnp.sum(xb * xb, axis=0)[:, None]
    d = (xsq + wsq_ref[...]) - mm2
    minval = jnp.min(d, axis=1, keepdims=True)
    eqm = d == minval
    oh_ref[...] = jnp.where(eqm, 1.0, 0.0)
    oh = oh_ref[...]

    ids = jax.lax.broadcasted_iota(jnp.int32, d.shape, 1)
    idx_ref[...] = jnp.min(jnp.where(eqm, ids, NUM_EMBEDDINGS), axis=1,
                           keepdims=True)

    cnt_ref[...] += jax.lax.dot_general(
        jnp.ones((8, _BLK), jnp.float32), oh, (((1,), (0,)), ((), ())),
        preferred_element_type=jnp.float32)
    acc_ref[0] += jnp.sum(minval)

    @pl.when(i == _NBLK - 1)
    def _():
        loss_ref[...] = jnp.full(
            (1, 1), (1.0 + COMMITMENT_COST) / (_ROWS * EMBEDDING_DIM)) \
            * acc_ref[0]
        p = cnt_ref[0:1, :] * (1.0 / _ROWS)
        perp_ref[...] = jnp.exp(
            -jnp.sum(p * jnp.log(p + 1e-10), keepdims=True))


def _dist_argmin(inputs, W2):
    return pl.pallas_call(
        _dist_argmin_kernel,
        grid=(_NBLK,),
        in_specs=[
            pl.BlockSpec((1, EMBEDDING_DIM, _BLK),
                         lambda i: (i // _TB, 0, i % _TB)),
            pl.BlockSpec((NUM_EMBEDDINGS, EMBEDDING_DIM), lambda i: (0, 0)),
        ],
        out_specs=[
            pl.BlockSpec((_BLK, 1), lambda i: (i, 0)),
            pl.BlockSpec((_BLK, NUM_EMBEDDINGS), lambda i: (i, 0)),
            pl.BlockSpec((1, 1), lambda i: (0, 0)),
            pl.BlockSpec((1, 1), lambda i: (0, 0)),
        ],
        out_shape=[
            jax.ShapeDtypeStruct((_ROWS, 1), jnp.int32),
            jax.ShapeDtypeStruct((_ROWS, NUM_EMBEDDINGS), jnp.float32),
            jax.ShapeDtypeStruct((1, 1), jnp.float32),
            jax.ShapeDtypeStruct((1, 1), jnp.float32),
        ],
        scratch_shapes=[
            pltpu.VMEM((1, NUM_EMBEDDINGS), jnp.float32),
            pltpu.VMEM((8, NUM_EMBEDDINGS), jnp.float32),
            pltpu.SMEM((1,), jnp.float32),
        ],
    )(inputs, W2)


def _sc_gather_body(idx_hbm, w_hbm, quant_hbm, idx_v, rows0, rows1, s0, s1):
    wid = lax.axis_index("s") * _NC + lax.axis_index("c")
    base = wid * _RPW
    pltpu.sync_copy(idx_hbm.at[wid], idx_v)
    rows = (rows0, rows1)
    sems = (s0, s1)
    copies = [None, None]
    copies[0] = pltpu.async_copy(w_hbm.at[idx_v.at[0]], rows0, s0)
    for c in range(_NCHUNK):
        if c + 1 < _NCHUNK:
            copies[(c + 1) % 2] = pltpu.async_copy(
                w_hbm.at[idx_v.at[c + 1]], rows[(c + 1) % 2],
                sems[(c + 1) % 2])
        copies[c % 2].wait()
        pltpu.sync_copy(rows[c % 2],
                        quant_hbm.at[pl.ds(base + c * _SUB, _SUB)])


def _sc_gather(encoding_indices, W):
    mesh = plsc.VectorSubcoreMesh(core_axis_name="c", subcore_axis_name="s")
    fn = functools.partial(
        pl.kernel,
        mesh=mesh,
        out_type=jax.ShapeDtypeStruct((_ROWS, EMBEDDING_DIM), jnp.float32),
        scratch_types=[
            pltpu.VMEM((_NCHUNK, _SUB), jnp.int32),
            pltpu.VMEM((_SUB, EMBEDDING_DIM), jnp.float32),
            pltpu.VMEM((_SUB, EMBEDDING_DIM), jnp.float32),
            pltpu.SemaphoreType.DMA,
            pltpu.SemaphoreType.DMA,
        ],
    )(_sc_gather_body)
    return fn(encoding_indices, W)


def _transpose_kernel(q_ref, out_ref):
    out_ref[0] = q_ref[...].T


def _transpose(quantized):
    return pl.pallas_call(
        _transpose_kernel,
        grid=(_N, _TB),
        in_specs=[
            pl.BlockSpec((_BLK, EMBEDDING_DIM),
                         lambda n, tb: (n * _TB + tb, 0)),
        ],
        out_specs=pl.BlockSpec((1, EMBEDDING_DIM, _BLK),
                               lambda n, tb: (n, 0, tb)),
        out_shape=jax.ShapeDtypeStruct((_N, EMBEDDING_DIM, _T), jnp.float32),
    )(quantized)


def kernel(inputs, W):
    W2 = W + W
    idx2, encodings, loss, perp = _dist_argmin(inputs, W2)
    idx_tiles = idx2.reshape(_NW, _NCHUNK, _SUB)

    quantized = _sc_gather(idx_tiles, W)
    quantized_out = _transpose(quantized)

    loss = loss.reshape(())
    perplexity = perp.reshape(())
    reset_ratio = jnp.zeros((1,), dtype=jnp.float32)

    return (loss, quantized_out, perplexity, reset_ratio, encodings)
